# Initial kernel scaffold; baseline (speedup 1.0000x reference)
import jax
import jax.numpy as jnp
from jax.experimental import pallas as pl


def kernel(x, edge_index, W1, a_src1, a_dst1, b1, W2, a_src2, a_dst2, b2):
    raise NotImplementedError("write your pallas kernel here")


# speedup vs baseline: 19.2371x; 19.2371x over previous
import functools

import jax
import jax.numpy as jnp
from jax import lax
from jax.experimental import pallas as pl
from jax.experimental.pallas import tpu as pltpu
from jax.experimental.pallas import tpu_sc as plsc

NN = 10000
EE = 320000
NEGS = 0.2

NC = 2
NS = 16
NW = NC * NS
EW = EE // NW
K = 100
NCH = EW // K
ROWS = 624

_mesh = plsc.VectorSubcoreMesh(core_axis_name="c", subcore_axis_name="s")


def _owned_copy(sid, mk_src, mk_dst):
    off = pl.multiple_of(sid * ROWS, 8)
    pltpu.sync_copy(mk_src(off, ROWS), mk_dst(off, ROWS))

    @pl.when(sid == NS - 1)
    def _():
        pltpu.sync_copy(mk_src(NS * ROWS, NN - NS * ROWS),
                        mk_dst(NS * ROWS, NN - NS * ROWS))


def _lrelu(x):
    return jnp.where(x >= 0, x, x * NEGS)


def _splat(v, lane):
    idx = jnp.full((16, 1), lane, jnp.int32)
    return lax.gather(
        v, idx,
        lax.GatherDimensionNumbers(offset_dims=(), collapsed_slice_dims=(0,),
                                   start_index_map=(0,)),
        (1,), mode=lax.GatherScatterMode.PROMISE_IN_BOUNDS)


@functools.partial(
    pl.kernel, mesh=_mesh,
    compiler_params=pltpu.CompilerParams(use_tc_tiling_on_sc=False),
    out_type=jax.ShapeDtypeStruct((NC, NN, 16), jnp.float32),
    scratch_types=[
        pltpu.VMEM((K,), jnp.int32),
        pltpu.VMEM((K,), jnp.int32),
        pltpu.VMEM((K, 16), jnp.float32),
        pltpu.VMEM((K, 16), jnp.float32),
        pltpu.VMEM((K, 16), jnp.float32),
        pltpu.VMEM((16,), jnp.float32),
        pltpu.VMEM((16,), jnp.float32),
        pltpu.VMEM_SHARED((NN, 16), jnp.float32),
        pltpu.SemaphoreType.DMA,
    ])
def _sc_stage1(src_h, dst_h, a_h, b_h, c_h, m_h, zden_h, den_out_h,
               src_v, dst_v, ra_v, rb_v, ex_v, c_v, m_v, den_sh, sem):
    cid = lax.axis_index("c")
    sid = lax.axis_index("s")
    wid = cid * NS + sid
    pltpu.sync_copy(c_h, c_v)
    pltpu.sync_copy(m_h, m_v)
    _owned_copy(sid, lambda o, s: zden_h.at[pl.ds(o, s)],
                lambda o, s: den_sh.at[pl.ds(o, s)])
    plsc.subcore_barrier()
    cvec = c_v[...]
    mvec = m_v[...]

    def chunk(ci, carry):
        pltpu.sync_copy(src_h.at[wid, ci], src_v)
        pltpu.sync_copy(dst_h.at[wid, ci], dst_v)
        pltpu.async_copy(a_h.at[dst_v], ra_v, sem).wait()
        pltpu.async_copy(b_h.at[src_v], rb_v, sem).wait()

        def edge(ei, c2):
            al = _lrelu(ra_v[ei] + rb_v[ei])
            ex_v[ei] = jnp.exp(al - cvec) * mvec
            return c2

        lax.fori_loop(0, K, edge, 0, unroll=4)
        pltpu.sync_copy(ex_v, den_sh.at[dst_v], add=True)
        return carry

    lax.fori_loop(0, NCH, chunk, 0)
    plsc.subcore_barrier()
    _owned_copy(sid, lambda o, s: den_sh.at[pl.ds(o, s)],
                lambda o, s: den_out_h.at[cid, pl.ds(o, s)])


def _make_sc_stage2(D, H):
    PER_H = D // H // 16

    @functools.partial(
        pl.kernel, mesh=_mesh,
        compiler_params=pltpu.CompilerParams(use_tc_tiling_on_sc=False),
        out_type=jax.ShapeDtypeStruct((NC, NN, D), jnp.float32),
        scratch_types=[
            pltpu.VMEM((K,), jnp.int32),
            pltpu.VMEM((K,), jnp.int32),
            pltpu.VMEM((K, 16), jnp.float32),
            pltpu.VMEM((K, 16), jnp.float32),
            pltpu.VMEM((K, 16), jnp.float32),
            pltpu.VMEM((K, D), jnp.float32),
            pltpu.VMEM((K, D), jnp.float32),
            pltpu.VMEM((16,), jnp.float32),
            pltpu.VMEM((16,), jnp.float32),
            pltpu.VMEM_SHARED((NN, D), jnp.float32),
            pltpu.SemaphoreType.DMA,
        ])
    def _sc_stage2(src_h, dst_h, a_h, b_h, c_h, m_h, den_h, feat_h, zout_h,
                   out_h, src_v, dst_v, ra_v, rb_v, rd_v, hr_v, wo_v,
                   c_v, m_v, out_sh, sem):
        cid = lax.axis_index("c")
        sid = lax.axis_index("s")
        wid = cid * NS + sid
        pltpu.sync_copy(c_h, c_v)
        pltpu.sync_copy(m_h, m_v)
        _owned_copy(sid, lambda o, s: zout_h.at[pl.ds(o, s)],
                    lambda o, s: out_sh.at[pl.ds(o, s)])
        plsc.subcore_barrier()
        cvec = c_v[...]
        mvec = m_v[...]

        def chunk(ci, carry):
            pltpu.sync_copy(src_h.at[wid, ci], src_v)
            pltpu.sync_copy(dst_h.at[wid, ci], dst_v)
            pltpu.async_copy(a_h.at[dst_v], ra_v, sem).wait()
            pltpu.async_copy(b_h.at[src_v], rb_v, sem).wait()
            pltpu.async_copy(den_h.at[dst_v], rd_v, sem).wait()
            pltpu.async_copy(feat_h.at[src_v], hr_v, sem).wait()

            def edge(ei, c2):
                al = _lrelu(ra_v[ei] + rb_v[ei])
                ex = jnp.exp(al - cvec) * mvec
                w = ex / (rd_v[ei] + 1e-16)
                for h in range(H):
                    wh = _splat(w, h)
                    for j in range(PER_H):
                        off = (h * PER_H + j) * 16
                        wo_v[ei, pl.ds(off, 16)] = (
                            hr_v[ei, pl.ds(off, 16)] * wh)
                return c2

            lax.fori_loop(0, K, edge, 0)
            pltpu.sync_copy(wo_v, out_sh.at[dst_v], add=True)
            return carry

        lax.fori_loop(0, NCH, chunk, 0)
        plsc.subcore_barrier()
        _owned_copy(sid, lambda o, s: out_sh.at[pl.ds(o, s)],
                    lambda o, s: out_h.at[cid, pl.ds(o, s)])

    return _sc_stage2


_sc_stage2_l1 = _make_sc_stage2(128, 8)
_sc_stage2_l2 = _make_sc_stage2(64, 1)


def _tc_linear(x, Wt, Wab):
    n, _ = x.shape
    dout = Wt.shape[1]

    def body(x_ref, w_ref, wab_ref, h_ref, s_ref, m_ref):
        h = jnp.dot(x_ref[...], w_ref[...], preferred_element_type=jnp.float32)
        h_ref[...] = h
        s = jnp.dot(h, wab_ref[...], preferred_element_type=jnp.float32)
        s_ref[...] = s
        m_ref[...] = jnp.max(s, axis=0, keepdims=True)

    return pl.pallas_call(
        body,
        out_shape=[jax.ShapeDtypeStruct((n, dout), jnp.float32),
                   jax.ShapeDtypeStruct((n, 32), jnp.float32),
                   jax.ShapeDtypeStruct((1, 32), jnp.float32)],
    )(x, Wt, Wab)


def _tc_combine_den(den_p, A, B, c_row, m_row):
    def body(p_ref, a_ref, b_ref, c_ref, m_ref, den_ref, w_ref):
        ex = jnp.exp(_lrelu(a_ref[...] + b_ref[...]) - c_ref[...]) * m_ref[...]
        den = p_ref[0] + p_ref[1] + ex
        den_ref[...] = den
        w_ref[...] = ex / (den + 1e-16)

    return pl.pallas_call(
        body,
        out_shape=[jax.ShapeDtypeStruct((NN, 16), jnp.float32),
                   jax.ShapeDtypeStruct((NN, 16), jnp.float32)],
    )(den_p, A, B, c_row, m_row)


def _tc_layer1_finish(out_p, wself, h1, b1_row, W2t, Wab2, R):
    def body(p_ref, w_ref, h_ref, b_ref, w2_ref, wab_ref, r_ref,
             h2_ref, s_ref, m_ref):
        wexp = jnp.dot(w_ref[...], r_ref[...],
                       preferred_element_type=jnp.float32)
        o = p_ref[0] + p_ref[1] + wexp * h_ref[...] + b_ref[...]
        o = jnp.where(o > 0, o, jnp.exp(o) - 1.0)
        h2 = jnp.dot(o, w2_ref[...], preferred_element_type=jnp.float32)
        h2_ref[...] = h2
        s = jnp.dot(h2, wab_ref[...], preferred_element_type=jnp.float32)
        s_ref[...] = s
        m_ref[...] = jnp.max(s, axis=0, keepdims=True)

    return pl.pallas_call(
        body,
        out_shape=[jax.ShapeDtypeStruct((NN, 64), jnp.float32),
                   jax.ShapeDtypeStruct((NN, 32), jnp.float32),
                   jax.ShapeDtypeStruct((1, 32), jnp.float32)],
    )(out_p, wself, h1, b1_row, W2t, Wab2, R)


def _tc_layer2_finish(out_p, wself2, h2, b2_row, R2):
    def body(p_ref, w_ref, h_ref, b_ref, r_ref, o_ref):
        wexp = jnp.dot(w_ref[...], r_ref[...],
                       preferred_element_type=jnp.float32)
        o = p_ref[0] + p_ref[1] + wexp * h_ref[...] + b_ref[...]
        m = jnp.max(o, axis=1, keepdims=True)
        e = jnp.exp(o - m)
        lse = m + jnp.log(jnp.sum(e, axis=1, keepdims=True))
        o_ref[...] = o - lse

    return pl.pallas_call(
        body,
        out_shape=jax.ShapeDtypeStruct((NN, 64), jnp.float32),
    )(out_p, wself2, h2, b2_row, R2)


def _pack_att(a_src, a_dst, heads, hid):
    eye = jnp.eye(heads, dtype=jnp.float32)
    msrc = (a_src.reshape(heads, hid)[:, :, None] *
            eye[:, None, :]).reshape(heads * hid, heads)
    mdst = (a_dst.reshape(heads, hid)[:, :, None] *
            eye[:, None, :]).reshape(heads * hid, heads)
    pad = jnp.zeros((heads * hid, 16 - heads), jnp.float32)
    return jnp.concatenate([msrc, pad, mdst, pad], axis=1)


def _expander(heads, width):
    r = jnp.zeros((16, heads * width), jnp.float32)
    for h in range(heads):
        r = r.at[h, h * width:(h + 1) * width].set(1.0)
    return r


@jax.jit
def kernel(x, edge_index, W1, a_src1, a_dst1, b1, W2, a_src2, a_dst2, b2):
    src3 = edge_index[0].reshape(NW, NCH, K)
    dst3 = edge_index[1].reshape(NW, NCH, K)
    z16 = jnp.zeros((NN, 16), jnp.float32)
    z128 = jnp.zeros((NN, 128), jnp.float32)
    z64 = jnp.zeros((NN, 64), jnp.float32)
    m1 = jnp.concatenate([jnp.ones(8, jnp.float32), jnp.zeros(8, jnp.float32)])
    m2 = jnp.concatenate([jnp.ones(1, jnp.float32), jnp.zeros(15, jnp.float32)])

    Wab1 = _pack_att(a_src1, a_dst1, 8, 16)
    h1, S1, M1 = _tc_linear(x, W1.T, Wab1)
    A1 = S1[:, :16]
    B1 = S1[:, 16:]
    c1 = _lrelu(M1[0, :16] + M1[0, 16:]) * m1
    den_p1 = _sc_stage1(src3, dst3, A1, B1, c1, m1, z16)
    den1, wself1 = _tc_combine_den(den_p1, A1, B1, c1.reshape(1, 16),
                                   m1.reshape(1, 16))
    out_p1 = _sc_stage2_l1(src3, dst3, A1, B1, c1, m1, den1, h1, z128)

    Wab2 = _pack_att(a_src2, a_dst2, 1, 64)
    h2, S2, M2 = _tc_layer1_finish(out_p1, wself1, h1, b1.reshape(1, 128),
                                   W2.T, Wab2, _expander(8, 16))
    A2 = S2[:, :16]
    B2 = S2[:, 16:]
    c2 = _lrelu(M2[0, :16] + M2[0, 16:]) * m2
    den_p2 = _sc_stage1(src3, dst3, A2, B2, c2, m2, z16)
    den2, wself2 = _tc_combine_den(den_p2, A2, B2, c2.reshape(1, 16),
                                   m2.reshape(1, 16))
    out_p2 = _sc_stage2_l2(src3, dst3, A2, B2, c2, m2, den2, h2, z64)

    return _tc_layer2_finish(out_p2, wself2, h2, b2.reshape(1, 64),
                             _expander(1, 64))


# speedup vs baseline: 37.4259x; 1.9455x over previous
import functools

import jax
import jax.numpy as jnp
from jax import lax
from jax.experimental import pallas as pl
from jax.experimental.pallas import tpu as pltpu
from jax.experimental.pallas import tpu_sc as plsc

NN = 10000
EE = 320000
NEGS = 0.2

NC = 2
NS = 16
NW = NC * NS
EW = EE // NW
K1 = 100
NCH1 = EW // K1
K2 = 40
NCH2 = EW // K2
ROWS = 624

_mesh = plsc.VectorSubcoreMesh(core_axis_name="c", subcore_axis_name="s")
_params = pltpu.CompilerParams(use_tc_tiling_on_sc=False)


def _owned_copy(sid, mk_src, mk_dst):
    off = pl.multiple_of(sid * ROWS, 8)
    pltpu.sync_copy(mk_src(off, ROWS), mk_dst(off, ROWS))

    @pl.when(sid == NS - 1)
    def _():
        pltpu.sync_copy(mk_src(NS * ROWS, NN - NS * ROWS),
                        mk_dst(NS * ROWS, NN - NS * ROWS))


def _lrelu(x):
    return jnp.where(x >= 0, x, x * NEGS)


def _splat(v, lane):
    idx = jnp.full((16, 1), lane, jnp.int32)
    return lax.gather(
        v, idx,
        lax.GatherDimensionNumbers(offset_dims=(), collapsed_slice_dims=(0,),
                                   start_index_map=(0,)),
        (1,), mode=lax.GatherScatterMode.PROMISE_IN_BOUNDS)


@functools.partial(
    pl.kernel, mesh=_mesh, compiler_params=_params,
    out_type=jax.ShapeDtypeStruct((NC, NN, 16), jnp.float32),
    scratch_types=[
        pltpu.VMEM((NCH1, K1), jnp.int32),
        pltpu.VMEM((NCH1, K1), jnp.int32),
        pltpu.VMEM((K1, 16), jnp.float32),
        pltpu.VMEM((K1, 16), jnp.float32),
        pltpu.VMEM((K1, 16), jnp.float32),
        pltpu.VMEM((K1, 16), jnp.float32),
        pltpu.VMEM((K1, 16), jnp.float32),
        pltpu.VMEM((K1, 16), jnp.float32),
        pltpu.VMEM((16,), jnp.float32),
        pltpu.VMEM((16,), jnp.float32),
        pltpu.VMEM_SHARED((NN, 16), jnp.float32),
        pltpu.SemaphoreType.DMA,
        pltpu.SemaphoreType.DMA,
        pltpu.SemaphoreType.DMA,
        pltpu.SemaphoreType.DMA,
    ])
def _sc_stage1(src_h, dst_h, a_h, b_h, c_h, m_h, zden_h, den_out_h,
               src_v, dst_v, ra0, rb0, ex0, ra1, rb1, ex1, c_v, m_v,
               den_sh, gsem0, gsem1, ssem0, ssem1):
    cid = lax.axis_index("c")
    sid = lax.axis_index("s")
    wid = cid * NS + sid
    pltpu.sync_copy(src_h.at[wid], src_v)
    pltpu.sync_copy(dst_h.at[wid], dst_v)
    pltpu.sync_copy(c_h, c_v)
    pltpu.sync_copy(m_h, m_v)
    _owned_copy(sid, lambda o, s: zden_h.at[pl.ds(o, s)],
                lambda o, s: den_sh.at[pl.ds(o, s)])
    plsc.subcore_barrier()
    cvec = c_v[...]
    mvec = m_v[...]
    bufs = ((ra0, rb0, ex0, gsem0, ssem0), (ra1, rb1, ex1, gsem1, ssem1))

    def gather_start(c, b):
        ra, rb, ex, gsem, ssem = bufs[b]
        pltpu.async_copy(a_h.at[dst_v.at[c]], ra, gsem)
        pltpu.async_copy(b_h.at[src_v.at[c]], rb, gsem)

    def gather_wait(c, b):
        ra, rb, ex, gsem, ssem = bufs[b]
        pltpu.make_async_copy(a_h.at[dst_v.at[c]], ra, gsem).wait()
        pltpu.make_async_copy(b_h.at[src_v.at[c]], rb, gsem).wait()

    def compute(b):
        ra, rb, ex, gsem, ssem = bufs[b]

        def edge(ei, c2):
            al = _lrelu(ra[ei] + rb[ei])
            ex[ei] = jnp.exp(al - cvec) * mvec
            return c2

        lax.fori_loop(0, K1, edge, 0, unroll=8)

    def scatter_start(c, b):
        ra, rb, ex, gsem, ssem = bufs[b]
        pltpu.async_copy(ex, den_sh.at[dst_v.at[c]], ssem, add=True)

    def scatter_wait(c, b):
        ra, rb, ex, gsem, ssem = bufs[b]
        pltpu.make_async_copy(ex, den_sh.at[dst_v.at[c]], ssem).wait()

    gather_start(0, 0)

    def pair(g, carry):
        c0 = 2 * g
        c1 = c0 + 1
        gather_start(c1, 1)
        gather_wait(c0, 0)

        @pl.when(g > 0)
        def _():
            scatter_wait(c0, 0)

        compute(0)
        scatter_start(c0, 0)

        @pl.when(g < NCH1 // 2 - 1)
        def _():
            gather_start(c0 + 2, 0)

        gather_wait(c1, 1)

        @pl.when(g > 0)
        def _():
            scatter_wait(c1, 1)

        compute(1)
        scatter_start(c1, 1)
        return carry

    lax.fori_loop(0, NCH1 // 2, pair, 0)
    scatter_wait(NCH1 - 2, 0)
    scatter_wait(NCH1 - 1, 1)
    plsc.subcore_barrier()
    _owned_copy(sid, lambda o, s: den_sh.at[pl.ds(o, s)],
                lambda o, s: den_out_h.at[cid, pl.ds(o, s)])


def _make_sc_stage2(D, H):
    PER_H = D // H // 16

    @functools.partial(
        pl.kernel, mesh=_mesh, compiler_params=_params,
        out_type=jax.ShapeDtypeStruct((NC, NN, D), jnp.float32),
        scratch_types=[
            pltpu.VMEM((NCH2, K2), jnp.int32),
            pltpu.VMEM((NCH2, K2), jnp.int32),
            pltpu.VMEM((K2, 32), jnp.float32),
            pltpu.VMEM((K2, D + 16), jnp.float32),
            pltpu.VMEM((K2, D), jnp.float32),
            pltpu.VMEM((K2, 32), jnp.float32),
            pltpu.VMEM((K2, D + 16), jnp.float32),
            pltpu.VMEM((K2, D), jnp.float32),
            pltpu.VMEM((16,), jnp.float32),
            pltpu.VMEM((16,), jnp.float32),
            pltpu.VMEM_SHARED((NN, D), jnp.float32),
            pltpu.SemaphoreType.DMA,
            pltpu.SemaphoreType.DMA,
            pltpu.SemaphoreType.DMA,
            pltpu.SemaphoreType.DMA,
        ])
    def _sc_stage2(src_h, dst_h, ad_h, fb_h, c_h, m_h, zout_h, out_h,
                   src_v, dst_v, rad0, rfb0, wo0, rad1, rfb1, wo1,
                   c_v, m_v, out_sh, gsem0, gsem1, ssem0, ssem1):
        cid = lax.axis_index("c")
        sid = lax.axis_index("s")
        wid = cid * NS + sid
        pltpu.sync_copy(src_h.at[wid], src_v)
        pltpu.sync_copy(dst_h.at[wid], dst_v)
        pltpu.sync_copy(c_h, c_v)
        pltpu.sync_copy(m_h, m_v)
        _owned_copy(sid, lambda o, s: zout_h.at[pl.ds(o, s)],
                    lambda o, s: out_sh.at[pl.ds(o, s)])
        plsc.subcore_barrier()
        cvec = c_v[...]
        mvec = m_v[...]
        bufs = ((rad0, rfb0, wo0, gsem0, ssem0),
                (rad1, rfb1, wo1, gsem1, ssem1))

        def gather_start(c, b):
            rad, rfb, wo, gsem, ssem = bufs[b]
            pltpu.async_copy(ad_h.at[dst_v.at[c]], rad, gsem)
            pltpu.async_copy(fb_h.at[src_v.at[c]], rfb, gsem)

        def gather_wait(c, b):
            rad, rfb, wo, gsem, ssem = bufs[b]
            pltpu.make_async_copy(ad_h.at[dst_v.at[c]], rad, gsem).wait()
            pltpu.make_async_copy(fb_h.at[src_v.at[c]], rfb, gsem).wait()

        def compute(b):
            rad, rfb, wo, gsem, ssem = bufs[b]

            def edge(ei, c2):
                va = rad[ei, pl.ds(0, 16)]
                dn = rad[ei, pl.ds(16, 16)]
                vb = rfb[ei, pl.ds(D, 16)]
                ex = jnp.exp(_lrelu(va + vb) - cvec) * mvec
                w = ex / (dn + 1e-16)
                for h in range(H):
                    wh = _splat(w, h)
                    for j in range(PER_H):
                        off = (h * PER_H + j) * 16
                        wo[ei, pl.ds(off, 16)] = rfb[ei, pl.ds(off, 16)] * wh
                return c2

            lax.fori_loop(0, K2, edge, 0, unroll=4)

        def scatter_start(c, b):
            rad, rfb, wo, gsem, ssem = bufs[b]
            pltpu.async_copy(wo, out_sh.at[dst_v.at[c]], ssem, add=True)

        def scatter_wait(c, b):
            rad, rfb, wo, gsem, ssem = bufs[b]
            pltpu.make_async_copy(wo, out_sh.at[dst_v.at[c]], ssem).wait()

        gather_start(0, 0)

        def pair(g, carry):
            c0 = 2 * g
            c1 = c0 + 1
            gather_start(c1, 1)
            gather_wait(c0, 0)

            @pl.when(g > 0)
            def _():
                scatter_wait(c0, 0)

            compute(0)
            scatter_start(c0, 0)

            @pl.when(g < NCH2 // 2 - 1)
            def _():
                gather_start(c0 + 2, 0)

            gather_wait(c1, 1)

            @pl.when(g > 0)
            def _():
                scatter_wait(c1, 1)

            compute(1)
            scatter_start(c1, 1)
            return carry

        lax.fori_loop(0, NCH2 // 2, pair, 0)
        scatter_wait(NCH2 - 2, 0)
        scatter_wait(NCH2 - 1, 1)
        plsc.subcore_barrier()
        _owned_copy(sid, lambda o, s: out_sh.at[pl.ds(o, s)],
                    lambda o, s: out_h.at[cid, pl.ds(o, s)])

    return _sc_stage2


_sc_stage2_l1 = _make_sc_stage2(128, 8)
_sc_stage2_l2 = _make_sc_stage2(64, 1)


def _tc_linear(x, Wt, Wab):
    n, _ = x.shape
    dout = Wt.shape[1]

    def body(x_ref, w_ref, wab_ref, h_ref, s_ref, m_ref):
        h = jnp.dot(x_ref[...], w_ref[...], preferred_element_type=jnp.float32)
        h_ref[...] = h
        s = jnp.dot(h, wab_ref[...], preferred_element_type=jnp.float32)
        s_ref[...] = s
        m_ref[...] = jnp.max(s, axis=0, keepdims=True)

    return pl.pallas_call(
        body,
        out_shape=[jax.ShapeDtypeStruct((n, dout), jnp.float32),
                   jax.ShapeDtypeStruct((n, 32), jnp.float32),
                   jax.ShapeDtypeStruct((1, 32), jnp.float32)],
    )(x, Wt, Wab)


def _tc_combine(den_p, A, B, c_row, m_row, feat):
    n, d = feat.shape

    def body(p_ref, a_ref, b_ref, c_ref, m_ref, f_ref, ad_ref, fb_ref, w_ref):
        a = a_ref[...]
        b = b_ref[...]
        ex = jnp.exp(_lrelu(a + b) - c_ref[...]) * m_ref[...]
        den = p_ref[0] + p_ref[1] + ex
        ad_ref[...] = jnp.concatenate([a, den], axis=1)
        fb_ref[...] = jnp.concatenate([f_ref[...], b], axis=1)
        w_ref[...] = ex / (den + 1e-16)

    return pl.pallas_call(
        body,
        out_shape=[jax.ShapeDtypeStruct((NN, 32), jnp.float32),
                   jax.ShapeDtypeStruct((NN, d + 16), jnp.float32),
                   jax.ShapeDtypeStruct((NN, 16), jnp.float32)],
    )(den_p, A, B, c_row, m_row, feat)


def _tc_layer1_finish(out_p, wself, h1, b1_row, W2t, Wab2, R):
    def body(p_ref, w_ref, h_ref, b_ref, w2_ref, wab_ref, r_ref,
             h2_ref, s_ref, m_ref):
        wexp = jnp.dot(w_ref[...], r_ref[...],
                       preferred_element_type=jnp.float32)
        o = p_ref[0] + p_ref[1] + wexp * h_ref[...] + b_ref[...]
        o = jnp.where(o > 0, o, jnp.exp(o) - 1.0)
        h2 = jnp.dot(o, w2_ref[...], preferred_element_type=jnp.float32)
        h2_ref[...] = h2
        s = jnp.dot(h2, wab_ref[...], preferred_element_type=jnp.float32)
        s_ref[...] = s
        m_ref[...] = jnp.max(s, axis=0, keepdims=True)

    return pl.pallas_call(
        body,
        out_shape=[jax.ShapeDtypeStruct((NN, 64), jnp.float32),
                   jax.ShapeDtypeStruct((NN, 32), jnp.float32),
                   jax.ShapeDtypeStruct((1, 32), jnp.float32)],
    )(out_p, wself, h1, b1_row, W2t, Wab2, R)


def _tc_layer2_finish(out_p, wself2, h2, b2_row, R2):
    def body(p_ref, w_ref, h_ref, b_ref, r_ref, o_ref):
        wexp = jnp.dot(w_ref[...], r_ref[...],
                       preferred_element_type=jnp.float32)
        o = p_ref[0] + p_ref[1] + wexp * h_ref[...] + b_ref[...]
        m = jnp.max(o, axis=1, keepdims=True)
        e = jnp.exp(o - m)
        lse = m + jnp.log(jnp.sum(e, axis=1, keepdims=True))
        o_ref[...] = o - lse

    return pl.pallas_call(
        body,
        out_shape=jax.ShapeDtypeStruct((NN, 64), jnp.float32),
    )(out_p, wself2, h2, b2_row, R2)


def _pack_att(a_src, a_dst, heads, hid):
    eye = jnp.eye(heads, dtype=jnp.float32)
    msrc = (a_src.reshape(heads, hid)[:, :, None] *
            eye[:, None, :]).reshape(heads * hid, heads)
    mdst = (a_dst.reshape(heads, hid)[:, :, None] *
            eye[:, None, :]).reshape(heads * hid, heads)
    pad = jnp.zeros((heads * hid, 16 - heads), jnp.float32)
    return jnp.concatenate([msrc, pad, mdst, pad], axis=1)


def _expander(heads, width):
    r = jnp.zeros((16, heads * width), jnp.float32)
    for h in range(heads):
        r = r.at[h, h * width:(h + 1) * width].set(1.0)
    return r


@jax.jit
def kernel(x, edge_index, W1, a_src1, a_dst1, b1, W2, a_src2, a_dst2, b2):
    src1 = edge_index[0].reshape(NW, NCH1, K1)
    dst1 = edge_index[1].reshape(NW, NCH1, K1)
    src2 = edge_index[0].reshape(NW, NCH2, K2)
    dst2 = edge_index[1].reshape(NW, NCH2, K2)
    z16 = jnp.zeros((NN, 16), jnp.float32)
    z128 = jnp.zeros((NN, 128), jnp.float32)
    z64 = jnp.zeros((NN, 64), jnp.float32)
    m1 = jnp.concatenate([jnp.ones(8, jnp.float32), jnp.zeros(8, jnp.float32)])
    m2 = jnp.concatenate([jnp.ones(1, jnp.float32), jnp.zeros(15, jnp.float32)])

    Wab1 = _pack_att(a_src1, a_dst1, 8, 16)
    h1, S1, M1 = _tc_linear(x, W1.T, Wab1)
    A1 = S1[:, :16]
    B1 = S1[:, 16:]
    c1 = _lrelu(M1[0, :16] + M1[0, 16:]) * m1
    den_p1 = _sc_stage1(src1, dst1, A1, B1, c1, m1, z16)
    ad1, fb1, wself1 = _tc_combine(den_p1, A1, B1, c1.reshape(1, 16),
                                   m1.reshape(1, 16), h1)
    out_p1 = _sc_stage2_l1(src2, dst2, ad1, fb1, c1, m1, z128)

    Wab2 = _pack_att(a_src2, a_dst2, 1, 64)
    h2, S2, M2 = _tc_layer1_finish(out_p1, wself1, h1, b1.reshape(1, 128),
                                   W2.T, Wab2, _expander(8, 16))
    A2 = S2[:, :16]
    B2 = S2[:, 16:]
    c2 = _lrelu(M2[0, :16] + M2[0, 16:]) * m2
    den_p2 = _sc_stage1(src1, dst1, A2, B2, c2, m2, z16)
    ad2, fb2, wself2 = _tc_combine(den_p2, A2, B2, c2.reshape(1, 16),
                                   m2.reshape(1, 16), h2)
    out_p2 = _sc_stage2_l2(src2, dst2, ad2, fb2, c2, m2, z64)

    return _tc_layer2_finish(out_p2, wself2, h2, b2.reshape(1, 64),
                             _expander(1, 64))


# speedup vs baseline: 37.4855x; 1.0016x over previous
import functools

import jax
import jax.numpy as jnp
from jax import lax
from jax.experimental import pallas as pl
from jax.experimental.pallas import tpu as pltpu
from jax.experimental.pallas import tpu_sc as plsc

NN = 10000
EE = 320000
NEGS = 0.2

NC = 2
NS = 16
NW = NC * NS
EW = EE // NW
K1 = 100
NCH1 = EW // K1
K2 = 40
NCH2 = EW // K2
ROWS = 624

_mesh = plsc.VectorSubcoreMesh(core_axis_name="c", subcore_axis_name="s")
_params = pltpu.CompilerParams(use_tc_tiling_on_sc=False)


def _owned_copy(sid, mk_src, mk_dst):
    off = pl.multiple_of(sid * ROWS, 8)
    pltpu.sync_copy(mk_src(off, ROWS), mk_dst(off, ROWS))

    @pl.when(sid == NS - 1)
    def _():
        pltpu.sync_copy(mk_src(NS * ROWS, NN - NS * ROWS),
                        mk_dst(NS * ROWS, NN - NS * ROWS))


def _lrelu(x):
    return jnp.where(x >= 0, x, x * NEGS)


def _splat(v, lane):
    idx = jnp.full((16, 1), lane, jnp.int32)
    return lax.gather(
        v, idx,
        lax.GatherDimensionNumbers(offset_dims=(), collapsed_slice_dims=(0,),
                                   start_index_map=(0,)),
        (1,), mode=lax.GatherScatterMode.PROMISE_IN_BOUNDS)


@functools.partial(
    pl.kernel, mesh=_mesh, compiler_params=_params,
    out_type=jax.ShapeDtypeStruct((NC, NN, 16), jnp.float32),
    scratch_types=[
        pltpu.VMEM((NCH1, K1), jnp.int32),
        pltpu.VMEM((NCH1, K1), jnp.int32),
        pltpu.VMEM((K1, 16), jnp.float32),
        pltpu.VMEM((K1, 16), jnp.float32),
        pltpu.VMEM((K1, 16), jnp.float32),
        pltpu.VMEM((K1, 16), jnp.float32),
        pltpu.VMEM((K1, 16), jnp.float32),
        pltpu.VMEM((K1, 16), jnp.float32),
        pltpu.VMEM((16,), jnp.float32),
        pltpu.VMEM((16,), jnp.float32),
        pltpu.VMEM_SHARED((NN, 16), jnp.float32),
        pltpu.SemaphoreType.DMA,
        pltpu.SemaphoreType.DMA,
        pltpu.SemaphoreType.DMA,
        pltpu.SemaphoreType.DMA,
    ])
def _sc_stage1(src_h, dst_h, a_h, b_h, c_h, m_h, zden_h, den_out_h,
               src_v, dst_v, ra0, rb0, ex0, ra1, rb1, ex1, c_v, m_v,
               den_sh, gsem0, gsem1, ssem0, ssem1):
    cid = lax.axis_index("c")
    sid = lax.axis_index("s")
    wid = cid * NS + sid
    pltpu.sync_copy(src_h.at[wid], src_v)
    pltpu.sync_copy(dst_h.at[wid], dst_v)
    pltpu.sync_copy(c_h, c_v)
    pltpu.sync_copy(m_h, m_v)
    _owned_copy(sid, lambda o, s: zden_h.at[pl.ds(o, s)],
                lambda o, s: den_sh.at[pl.ds(o, s)])
    plsc.subcore_barrier()
    cvec = c_v[...]
    mvec = m_v[...]
    bufs = ((ra0, rb0, ex0, gsem0, ssem0), (ra1, rb1, ex1, gsem1, ssem1))

    def gather_start(c, b):
        ra, rb, ex, gsem, ssem = bufs[b]
        pltpu.async_copy(a_h.at[dst_v.at[c]], ra, gsem)
        pltpu.async_copy(b_h.at[src_v.at[c]], rb, gsem)

    def gather_wait(c, b):
        ra, rb, ex, gsem, ssem = bufs[b]
        pltpu.make_async_copy(a_h.at[dst_v.at[c]], ra, gsem).wait()
        pltpu.make_async_copy(b_h.at[src_v.at[c]], rb, gsem).wait()

    def compute(b):
        ra, rb, ex, gsem, ssem = bufs[b]

        def edge(ei, c2):
            al = _lrelu(ra[ei] + rb[ei])
            ex[ei] = jnp.exp(al - cvec) * mvec
            return c2

        lax.fori_loop(0, K1, edge, 0, unroll=8)

    def scatter_start(c, b):
        ra, rb, ex, gsem, ssem = bufs[b]
        pltpu.async_copy(ex, den_sh.at[dst_v.at[c]], ssem, add=True)

    def scatter_wait(c, b):
        ra, rb, ex, gsem, ssem = bufs[b]
        pltpu.make_async_copy(ex, den_sh.at[dst_v.at[c]], ssem).wait()

    gather_start(0, 0)

    def pair(g, carry):
        c0 = 2 * g
        c1 = c0 + 1
        gather_start(c1, 1)
        gather_wait(c0, 0)

        @pl.when(g > 0)
        def _():
            scatter_wait(c0, 0)

        compute(0)
        scatter_start(c0, 0)

        @pl.when(g < NCH1 // 2 - 1)
        def _():
            gather_start(c0 + 2, 0)

        gather_wait(c1, 1)

        @pl.when(g > 0)
        def _():
            scatter_wait(c1, 1)

        compute(1)
        scatter_start(c1, 1)
        return carry

    lax.fori_loop(0, NCH1 // 2, pair, 0)
    scatter_wait(NCH1 - 2, 0)
    scatter_wait(NCH1 - 1, 1)
    plsc.subcore_barrier()
    _owned_copy(sid, lambda o, s: den_sh.at[pl.ds(o, s)],
                lambda o, s: den_out_h.at[cid, pl.ds(o, s)])


def _make_sc_stage2(D, H):
    PER_H = D // H // 16

    @functools.partial(
        pl.kernel, mesh=_mesh, compiler_params=_params,
        out_type=jax.ShapeDtypeStruct((NC, NN, D), jnp.float32),
        scratch_types=[
            pltpu.VMEM((NCH2, K2), jnp.int32),
            pltpu.VMEM((NCH2, K2), jnp.int32),
            pltpu.VMEM((K2, 32), jnp.float32),
            pltpu.VMEM((K2, D + 16), jnp.float32),
            pltpu.VMEM((K2, D), jnp.float32),
            pltpu.VMEM((K2, 32), jnp.float32),
            pltpu.VMEM((K2, D + 16), jnp.float32),
            pltpu.VMEM((K2, D), jnp.float32),
            pltpu.VMEM((16,), jnp.float32),
            pltpu.VMEM((16,), jnp.float32),
            pltpu.VMEM_SHARED((NN, D), jnp.float32),
            pltpu.SemaphoreType.DMA,
            pltpu.SemaphoreType.DMA,
            pltpu.SemaphoreType.DMA,
            pltpu.SemaphoreType.DMA,
        ])
    def _sc_stage2(src_h, dst_h, ad_h, fb_h, c_h, m_h, zout_h, out_h,
                   src_v, dst_v, rad0, rfb0, wo0, rad1, rfb1, wo1,
                   c_v, m_v, out_sh, gsem0, gsem1, ssem0, ssem1):
        cid = lax.axis_index("c")
        sid = lax.axis_index("s")
        wid = cid * NS + sid
        pltpu.sync_copy(src_h.at[wid], src_v)
        pltpu.sync_copy(dst_h.at[wid], dst_v)
        pltpu.sync_copy(c_h, c_v)
        pltpu.sync_copy(m_h, m_v)
        _owned_copy(sid, lambda o, s: zout_h.at[pl.ds(o, s)],
                    lambda o, s: out_sh.at[pl.ds(o, s)])
        plsc.subcore_barrier()
        cvec = c_v[...]
        mvec = m_v[...]
        bufs = ((rad0, rfb0, wo0, gsem0, ssem0),
                (rad1, rfb1, wo1, gsem1, ssem1))

        def gather_start(c, b):
            rad, rfb, wo, gsem, ssem = bufs[b]
            pltpu.async_copy(ad_h.at[dst_v.at[c]], rad, gsem)
            pltpu.async_copy(fb_h.at[src_v.at[c]], rfb, gsem)

        def gather_wait(c, b):
            rad, rfb, wo, gsem, ssem = bufs[b]
            pltpu.make_async_copy(ad_h.at[dst_v.at[c]], rad, gsem).wait()
            pltpu.make_async_copy(fb_h.at[src_v.at[c]], rfb, gsem).wait()

        def compute(b):
            rad, rfb, wo, gsem, ssem = bufs[b]

            def edge(ei, c2):
                va = rad[ei, pl.ds(0, 16)]
                dn = rad[ei, pl.ds(16, 16)]
                vb = rfb[ei, pl.ds(D, 16)]
                ex = jnp.exp(_lrelu(va + vb) - cvec) * mvec
                w = ex / (dn + 1e-16)
                for h in range(H):
                    wh = _splat(w, h)
                    for j in range(PER_H):
                        off = (h * PER_H + j) * 16
                        wo[ei, pl.ds(off, 16)] = rfb[ei, pl.ds(off, 16)] * wh
                return c2

            lax.fori_loop(0, K2, edge, 0, unroll=8)

        def scatter_start(c, b):
            rad, rfb, wo, gsem, ssem = bufs[b]
            pltpu.async_copy(wo, out_sh.at[dst_v.at[c]], ssem, add=True)

        def scatter_wait(c, b):
            rad, rfb, wo, gsem, ssem = bufs[b]
            pltpu.make_async_copy(wo, out_sh.at[dst_v.at[c]], ssem).wait()

        gather_start(0, 0)

        def pair(g, carry):
            c0 = 2 * g
            c1 = c0 + 1
            gather_start(c1, 1)
            gather_wait(c0, 0)

            @pl.when(g > 0)
            def _():
                scatter_wait(c0, 0)

            compute(0)
            scatter_start(c0, 0)

            @pl.when(g < NCH2 // 2 - 1)
            def _():
                gather_start(c0 + 2, 0)

            gather_wait(c1, 1)

            @pl.when(g > 0)
            def _():
                scatter_wait(c1, 1)

            compute(1)
            scatter_start(c1, 1)
            return carry

        lax.fori_loop(0, NCH2 // 2, pair, 0)
        scatter_wait(NCH2 - 2, 0)
        scatter_wait(NCH2 - 1, 1)
        plsc.subcore_barrier()
        _owned_copy(sid, lambda o, s: out_sh.at[pl.ds(o, s)],
                    lambda o, s: out_h.at[cid, pl.ds(o, s)])

    return _sc_stage2


_sc_stage2_l1 = _make_sc_stage2(128, 8)
_sc_stage2_l2 = _make_sc_stage2(64, 1)


def _tc_linear(x, Wt, Wab):
    n, _ = x.shape
    dout = Wt.shape[1]

    def body(x_ref, w_ref, wab_ref, h_ref, s_ref, m_ref):
        h = jnp.dot(x_ref[...], w_ref[...], preferred_element_type=jnp.float32)
        h_ref[...] = h
        s = jnp.dot(h, wab_ref[...], preferred_element_type=jnp.float32)
        s_ref[...] = s
        m_ref[...] = jnp.max(s, axis=0, keepdims=True)

    return pl.pallas_call(
        body,
        out_shape=[jax.ShapeDtypeStruct((n, dout), jnp.float32),
                   jax.ShapeDtypeStruct((n, 32), jnp.float32),
                   jax.ShapeDtypeStruct((1, 32), jnp.float32)],
    )(x, Wt, Wab)


def _tc_combine(den_p, A, B, c_row, m_row, feat):
    n, d = feat.shape

    def body(p_ref, a_ref, b_ref, c_ref, m_ref, f_ref, ad_ref, fb_ref, w_ref):
        a = a_ref[...]
        b = b_ref[...]
        ex = jnp.exp(_lrelu(a + b) - c_ref[...]) * m_ref[...]
        den = p_ref[0] + p_ref[1] + ex
        ad_ref[...] = jnp.concatenate([a, den], axis=1)
        fb_ref[...] = jnp.concatenate([f_ref[...], b], axis=1)
        w_ref[...] = ex / (den + 1e-16)

    return pl.pallas_call(
        body,
        out_shape=[jax.ShapeDtypeStruct((NN, 32), jnp.float32),
                   jax.ShapeDtypeStruct((NN, d + 16), jnp.float32),
                   jax.ShapeDtypeStruct((NN, 16), jnp.float32)],
    )(den_p, A, B, c_row, m_row, feat)


def _tc_layer1_finish(out_p, wself, h1, b1_row, W2t, Wab2, R):
    def body(p_ref, w_ref, h_ref, b_ref, w2_ref, wab_ref, r_ref,
             h2_ref, s_ref, m_ref):
        wexp = jnp.dot(w_ref[...], r_ref[...],
                       preferred_element_type=jnp.float32)
        o = p_ref[0] + p_ref[1] + wexp * h_ref[...] + b_ref[...]
        o = jnp.where(o > 0, o, jnp.exp(o) - 1.0)
        h2 = jnp.dot(o, w2_ref[...], preferred_element_type=jnp.float32)
        h2_ref[...] = h2
        s = jnp.dot(h2, wab_ref[...], preferred_element_type=jnp.float32)
        s_ref[...] = s
        m_ref[...] = jnp.max(s, axis=0, keepdims=True)

    return pl.pallas_call(
        body,
        out_shape=[jax.ShapeDtypeStruct((NN, 64), jnp.float32),
                   jax.ShapeDtypeStruct((NN, 32), jnp.float32),
                   jax.ShapeDtypeStruct((1, 32), jnp.float32)],
    )(out_p, wself, h1, b1_row, W2t, Wab2, R)


def _tc_layer2_finish(out_p, wself2, h2, b2_row, R2):
    def body(p_ref, w_ref, h_ref, b_ref, r_ref, o_ref):
        wexp = jnp.dot(w_ref[...], r_ref[...],
                       preferred_element_type=jnp.float32)
        o = p_ref[0] + p_ref[1] + wexp * h_ref[...] + b_ref[...]
        m = jnp.max(o, axis=1, keepdims=True)
        e = jnp.exp(o - m)
        lse = m + jnp.log(jnp.sum(e, axis=1, keepdims=True))
        o_ref[...] = o - lse

    return pl.pallas_call(
        body,
        out_shape=jax.ShapeDtypeStruct((NN, 64), jnp.float32),
    )(out_p, wself2, h2, b2_row, R2)


def _pack_att(a_src, a_dst, heads, hid):
    eye = jnp.eye(heads, dtype=jnp.float32)
    msrc = (a_src.reshape(heads, hid)[:, :, None] *
            eye[:, None, :]).reshape(heads * hid, heads)
    mdst = (a_dst.reshape(heads, hid)[:, :, None] *
            eye[:, None, :]).reshape(heads * hid, heads)
    pad = jnp.zeros((heads * hid, 16 - heads), jnp.float32)
    return jnp.concatenate([msrc, pad, mdst, pad], axis=1)


def _expander(heads, width):
    r = jnp.zeros((16, heads * width), jnp.float32)
    for h in range(heads):
        r = r.at[h, h * width:(h + 1) * width].set(1.0)
    return r


@jax.jit
def kernel(x, edge_index, W1, a_src1, a_dst1, b1, W2, a_src2, a_dst2, b2):
    src1 = edge_index[0].reshape(NW, NCH1, K1)
    dst1 = edge_index[1].reshape(NW, NCH1, K1)
    src2 = edge_index[0].reshape(NW, NCH2, K2)
    dst2 = edge_index[1].reshape(NW, NCH2, K2)
    z16 = jnp.zeros((NN, 16), jnp.float32)
    z128 = jnp.zeros((NN, 128), jnp.float32)
    z64 = jnp.zeros((NN, 64), jnp.float32)
    m1 = jnp.concatenate([jnp.ones(8, jnp.float32), jnp.zeros(8, jnp.float32)])
    m2 = jnp.concatenate([jnp.ones(1, jnp.float32), jnp.zeros(15, jnp.float32)])

    Wab1 = _pack_att(a_src1, a_dst1, 8, 16)
    h1, S1, M1 = _tc_linear(x, W1.T, Wab1)
    A1 = S1[:, :16]
    B1 = S1[:, 16:]
    c1 = _lrelu(M1[0, :16] + M1[0, 16:]) * m1
    den_p1 = _sc_stage1(src1, dst1, A1, B1, c1, m1, z16)
    ad1, fb1, wself1 = _tc_combine(den_p1, A1, B1, c1.reshape(1, 16),
                                   m1.reshape(1, 16), h1)
    out_p1 = _sc_stage2_l1(src2, dst2, ad1, fb1, c1, m1, z128)

    Wab2 = _pack_att(a_src2, a_dst2, 1, 64)
    h2, S2, M2 = _tc_layer1_finish(out_p1, wself1, h1, b1.reshape(1, 128),
                                   W2.T, Wab2, _expander(8, 16))
    A2 = S2[:, :16]
    B2 = S2[:, 16:]
    c2 = _lrelu(M2[0, :16] + M2[0, 16:]) * m2
    den_p2 = _sc_stage1(src1, dst1, A2, B2, c2, m2, z16)
    ad2, fb2, wself2 = _tc_combine(den_p2, A2, B2, c2.reshape(1, 16),
                                   m2.reshape(1, 16), h2)
    out_p2 = _sc_stage2_l2(src2, dst2, ad2, fb2, c2, m2, z64)

    return _tc_layer2_finish(out_p2, wself2, h2, b2.reshape(1, 64),
                             _expander(1, 64))


# speedup vs baseline: 48.1137x; 1.2835x over previous
import functools

import jax
import jax.numpy as jnp
from jax import lax
from jax.experimental import pallas as pl
from jax.experimental.pallas import tpu as pltpu
from jax.experimental.pallas import tpu_sc as plsc

NN = 10000
EE = 320000
NEGS = 0.2

NC = 2
NS = 16
NW = NC * NS
EW = EE // NW
K = 25
NCH = EW // K
ROWS = 624

_mesh = plsc.VectorSubcoreMesh(core_axis_name="c", subcore_axis_name="s")
_params = pltpu.CompilerParams(use_tc_tiling_on_sc=False)


def _owned_copy(sid, mk_src, mk_dst):
    off = pl.multiple_of(sid * ROWS, 8)
    pltpu.sync_copy(mk_src(off, ROWS), mk_dst(off, ROWS))

    @pl.when(sid == NS - 1)
    def _():
        pltpu.sync_copy(mk_src(NS * ROWS, NN - NS * ROWS),
                        mk_dst(NS * ROWS, NN - NS * ROWS))


def _lrelu(x):
    return jnp.where(x >= 0, x, x * NEGS)


def _splat(v, lane):
    idx = jnp.full((16, 1), lane, jnp.int32)
    return lax.gather(
        v, idx,
        lax.GatherDimensionNumbers(offset_dims=(), collapsed_slice_dims=(0,),
                                   start_index_map=(0,)),
        (1,), mode=lax.GatherScatterMode.PROMISE_IN_BOUNDS)


def _make_sc_edge(D, H):
    PER_H = D // H // 16
    W = D + 16

    @functools.partial(
        pl.kernel, mesh=_mesh, compiler_params=_params,
        out_type=jax.ShapeDtypeStruct((NC, NN, W), jnp.float32),
        scratch_types=[
            pltpu.VMEM((NCH, K), jnp.int32),
            pltpu.VMEM((NCH, K), jnp.int32),
            pltpu.VMEM((K, 16), jnp.float32),
            pltpu.VMEM((K, W), jnp.float32),
            pltpu.VMEM((K, W), jnp.float32),
            pltpu.VMEM((K, 16), jnp.float32),
            pltpu.VMEM((K, W), jnp.float32),
            pltpu.VMEM((K, W), jnp.float32),
            pltpu.VMEM((16,), jnp.float32),
            pltpu.VMEM((16,), jnp.float32),
            pltpu.VMEM_SHARED((NN, W), jnp.float32),
            pltpu.SemaphoreType.DMA,
            pltpu.SemaphoreType.DMA,
            pltpu.SemaphoreType.DMA,
            pltpu.SemaphoreType.DMA,
        ])
    def _sc_edge(src_h, dst_h, a_h, fb_h, c_h, m_h, zacc_h, out_h,
                 src_v, dst_v, ra0, rfb0, wo0, ra1, rfb1, wo1,
                 c_v, m_v, acc_sh, gsem0, gsem1, ssem0, ssem1):
        cid = lax.axis_index("c")
        sid = lax.axis_index("s")
        wid = cid * NS + sid
        pltpu.sync_copy(src_h.at[wid], src_v)
        pltpu.sync_copy(dst_h.at[wid], dst_v)
        pltpu.sync_copy(c_h, c_v)
        pltpu.sync_copy(m_h, m_v)
        _owned_copy(sid, lambda o, s: zacc_h.at[pl.ds(o, s)],
                    lambda o, s: acc_sh.at[pl.ds(o, s)])
        plsc.subcore_barrier()
        cvec = c_v[...]
        mvec = m_v[...]
        bufs = ((ra0, rfb0, wo0, gsem0, ssem0),
                (ra1, rfb1, wo1, gsem1, ssem1))

        def gather_start(c, b):
            ra, rfb, wo, gsem, ssem = bufs[b]
            pltpu.async_copy(a_h.at[dst_v.at[c]], ra, gsem)
            pltpu.async_copy(fb_h.at[src_v.at[c]], rfb, gsem)

        def gather_wait(c, b):
            ra, rfb, wo, gsem, ssem = bufs[b]
            pltpu.make_async_copy(a_h.at[dst_v.at[c]], ra, gsem).wait()
            pltpu.make_async_copy(fb_h.at[src_v.at[c]], rfb, gsem).wait()

        def compute(b):
            ra, rfb, wo, gsem, ssem = bufs[b]

            def edge(ei, c2):
                va = ra[ei]
                vb = rfb[ei, pl.ds(D, 16)]
                ex = jnp.exp(_lrelu(va + vb) - cvec) * mvec
                wo[ei, pl.ds(D, 16)] = ex
                for h in range(H):
                    eh = _splat(ex, h)
                    for j in range(PER_H):
                        off = (h * PER_H + j) * 16
                        wo[ei, pl.ds(off, 16)] = rfb[ei, pl.ds(off, 16)] * eh
                return c2

            lax.fori_loop(0, K, edge, 0, unroll=5)

        def scatter_start(c, b):
            ra, rfb, wo, gsem, ssem = bufs[b]
            pltpu.async_copy(wo, acc_sh.at[dst_v.at[c]], ssem, add=True)

        def scatter_wait(c, b):
            ra, rfb, wo, gsem, ssem = bufs[b]
            pltpu.make_async_copy(wo, acc_sh.at[dst_v.at[c]], ssem).wait()

        gather_start(0, 0)

        def pair(g, carry):
            c0 = 2 * g
            c1 = c0 + 1
            gather_start(c1, 1)
            gather_wait(c0, 0)

            @pl.when(g > 0)
            def _():
                scatter_wait(c0, 0)

            compute(0)
            scatter_start(c0, 0)

            @pl.when(g < NCH // 2 - 1)
            def _():
                gather_start(c0 + 2, 0)

            gather_wait(c1, 1)

            @pl.when(g > 0)
            def _():
                scatter_wait(c1, 1)

            compute(1)
            scatter_start(c1, 1)
            return carry

        lax.fori_loop(0, NCH // 2, pair, 0)
        scatter_wait(NCH - 2, 0)
        scatter_wait(NCH - 1, 1)
        plsc.subcore_barrier()
        _owned_copy(sid, lambda o, s: acc_sh.at[pl.ds(o, s)],
                    lambda o, s: out_h.at[cid, pl.ds(o, s)])

    return _sc_edge


_sc_edge_l1 = _make_sc_edge(128, 8)
_sc_edge_l2 = _make_sc_edge(64, 1)


def _tc_linear(x, Wt, Wab):
    n, _ = x.shape
    dout = Wt.shape[1]

    def body(x_ref, w_ref, wab_ref, s_ref, m_ref, fb_ref):
        h = jnp.dot(x_ref[...], w_ref[...], preferred_element_type=jnp.float32)
        s = jnp.dot(h, wab_ref[...], preferred_element_type=jnp.float32)
        s_ref[...] = s
        m_ref[...] = jnp.max(s, axis=0, keepdims=True)
        fb_ref[...] = jnp.concatenate([h, s[:, 16:]], axis=1)

    return pl.pallas_call(
        body,
        out_shape=[jax.ShapeDtypeStruct((n, 32), jnp.float32),
                   jax.ShapeDtypeStruct((1, 32), jnp.float32),
                   jax.ShapeDtypeStruct((n, dout + 16), jnp.float32)],
    )(x, Wt, Wab)


def _tc_mid(p, S1, c_row, m_row, b1_row, W2t, Wab2, R, fb1):
    def body(p_ref, s1_ref, c_ref, m_ref, b_ref, w2_ref, wab_ref, r_ref,
             fb1_ref, s_ref, m2_ref, fb_ref):
        s1 = s1_ref[...]
        a = s1[:, :16]
        b = s1[:, 16:]
        ex = jnp.exp(_lrelu(a + b) - c_ref[...]) * m_ref[...]
        acc = p_ref[0] + p_ref[1]
        den16 = acc[:, 128:] + ex
        exx = jnp.dot(ex, r_ref[...], preferred_element_type=jnp.float32)
        denx = jnp.dot(den16, r_ref[...], preferred_element_type=jnp.float32)
        h1 = fb1_ref[...][:, :128]
        nom = acc[:, :128] + exx * h1
        o = nom / (denx + 1e-16) + b_ref[...]
        o = jnp.where(o > 0, o, jnp.exp(o) - 1.0)
        h2 = jnp.dot(o, w2_ref[...], preferred_element_type=jnp.float32)
        s2 = jnp.dot(h2, wab_ref[...], preferred_element_type=jnp.float32)
        s_ref[...] = s2
        m2_ref[...] = jnp.max(s2, axis=0, keepdims=True)
        fb_ref[...] = jnp.concatenate([h2, s2[:, 16:]], axis=1)

    return pl.pallas_call(
        body,
        out_shape=[jax.ShapeDtypeStruct((NN, 32), jnp.float32),
                   jax.ShapeDtypeStruct((1, 32), jnp.float32),
                   jax.ShapeDtypeStruct((NN, 80), jnp.float32)],
    )(p, S1, c_row, m_row, b1_row, W2t, Wab2, R, fb1)


def _tc_final(p, S2, c_row, m_row, fb2, b2_row, R2):
    def body(p_ref, s2_ref, c_ref, m_ref, fb_ref, b_ref, r_ref, o_ref):
        s2 = s2_ref[...]
        a = s2[:, :16]
        b = s2[:, 16:]
        ex = jnp.exp(_lrelu(a + b) - c_ref[...]) * m_ref[...]
        acc = p_ref[0] + p_ref[1]
        den16 = acc[:, 64:] + ex
        exx = jnp.dot(ex, r_ref[...], preferred_element_type=jnp.float32)
        denx = jnp.dot(den16, r_ref[...], preferred_element_type=jnp.float32)
        h2 = fb_ref[...][:, :64]
        nom = acc[:, :64] + exx * h2
        o = nom / (denx + 1e-16) + b_ref[...]
        m = jnp.max(o, axis=1, keepdims=True)
        e = jnp.exp(o - m)
        lse = m + jnp.log(jnp.sum(e, axis=1, keepdims=True))
        o_ref[...] = o - lse

    return pl.pallas_call(
        body,
        out_shape=jax.ShapeDtypeStruct((NN, 64), jnp.float32),
    )(p, S2, c_row, m_row, fb2, b2_row, R2)


def _pack_att(a_src, a_dst, heads, hid):
    eye = jnp.eye(heads, dtype=jnp.float32)
    msrc = (a_src.reshape(heads, hid)[:, :, None] *
            eye[:, None, :]).reshape(heads * hid, heads)
    mdst = (a_dst.reshape(heads, hid)[:, :, None] *
            eye[:, None, :]).reshape(heads * hid, heads)
    pad = jnp.zeros((heads * hid, 16 - heads), jnp.float32)
    return jnp.concatenate([msrc, pad, mdst, pad], axis=1)


def _expander(heads, width):
    r = jnp.zeros((16, heads * width), jnp.float32)
    for h in range(heads):
        r = r.at[h, h * width:(h + 1) * width].set(1.0)
    return r


@jax.jit
def kernel(x, edge_index, W1, a_src1, a_dst1, b1, W2, a_src2, a_dst2, b2):
    src3 = edge_index[0].reshape(NW, NCH, K)
    dst3 = edge_index[1].reshape(NW, NCH, K)
    z144 = jnp.zeros((NN, 144), jnp.float32)
    z80 = jnp.zeros((NN, 80), jnp.float32)
    m1 = jnp.concatenate([jnp.ones(8, jnp.float32), jnp.zeros(8, jnp.float32)])
    m2 = jnp.concatenate([jnp.ones(1, jnp.float32), jnp.zeros(15, jnp.float32)])

    Wab1 = _pack_att(a_src1, a_dst1, 8, 16)
    S1, M1, fb1 = _tc_linear(x, W1.T, Wab1)
    A1 = S1[:, :16]
    c1 = _lrelu(M1[0, :16] + M1[0, 16:]) * m1
    p1 = _sc_edge_l1(src3, dst3, A1, fb1, c1, m1, z144)

    Wab2 = _pack_att(a_src2, a_dst2, 1, 64)
    S2, M2, fb2 = _tc_mid(p1, S1, c1.reshape(1, 16), m1.reshape(1, 16),
                          b1.reshape(1, 128), W2.T, Wab2, _expander(8, 16),
                          fb1)
    A2 = S2[:, :16]
    c2 = _lrelu(M2[0, :16] + M2[0, 16:]) * m2
    p2 = _sc_edge_l2(src3, dst3, A2, fb2, c2, m2, z80)

    return _tc_final(p2, S2, c2.reshape(1, 16), m2.reshape(1, 16), fb2,
                     b2.reshape(1, 64), _expander(1, 64))


# speedup vs baseline: 89.1139x; 1.8522x over previous
import functools

import jax
import jax.numpy as jnp
from jax import lax
from jax.experimental import pallas as pl
from jax.experimental.pallas import tpu as pltpu
from jax.experimental.pallas import tpu_sc as plsc

NN = 10000
EE = 320000
NEGS = 0.2

NC = 2
NS = 16
NW = NC * NS
EW = EE // NW
K = 25
NCH = EW // K
ROWS = 624

_mesh = plsc.VectorSubcoreMesh(core_axis_name="c", subcore_axis_name="s")
_params = pltpu.CompilerParams(use_tc_tiling_on_sc=False)


def _owned_copy(sid, mk_src, mk_dst):
    off = pl.multiple_of(sid * ROWS, 8)
    pltpu.sync_copy(mk_src(off, ROWS), mk_dst(off, ROWS))

    @pl.when(sid == NS - 1)
    def _():
        pltpu.sync_copy(mk_src(NS * ROWS, NN - NS * ROWS),
                        mk_dst(NS * ROWS, NN - NS * ROWS))


def _lrelu(x):
    return jnp.where(x >= 0, x, x * NEGS)


def _splat(v, lane):
    idx = jnp.full((16, 1), lane, jnp.int32)
    return lax.gather(
        v, idx,
        lax.GatherDimensionNumbers(offset_dims=(), collapsed_slice_dims=(0,),
                                   start_index_map=(0,)),
        (1,), mode=lax.GatherScatterMode.PROMISE_IN_BOUNDS)


def _make_sc_edge(D, H):
    PER_H = D // H // 16
    W = D + 16

    @functools.partial(
        pl.kernel, mesh=_mesh, compiler_params=_params,
        out_type=jax.ShapeDtypeStruct((NC, NN, W), jnp.float32),
        scratch_types=[
            pltpu.VMEM((NCH, K), jnp.int32),
            pltpu.VMEM((NCH, K), jnp.int32),
            pltpu.VMEM((K, 16), jnp.float32),
            pltpu.VMEM((K, W), jnp.float32),
            pltpu.VMEM((K, W), jnp.float32),
            pltpu.VMEM((K, 16), jnp.float32),
            pltpu.VMEM((K, W), jnp.float32),
            pltpu.VMEM((K, W), jnp.float32),
            pltpu.VMEM((16,), jnp.float32),
            pltpu.VMEM((16,), jnp.float32),
            pltpu.VMEM_SHARED((NN, W), jnp.float32),
            pltpu.SemaphoreType.DMA,
            pltpu.SemaphoreType.DMA,
            pltpu.SemaphoreType.DMA,
            pltpu.SemaphoreType.DMA,
        ])
    def _sc_edge(src_h, dst_h, a_h, fb_h, c_h, m_h, zacc_h, out_h,
                 src_v, dst_v, ra0, rfb0, wo0, ra1, rfb1, wo1,
                 c_v, m_v, acc_sh, gsem0, gsem1, ssem0, ssem1):
        cid = lax.axis_index("c")
        sid = lax.axis_index("s")
        wid = cid * NS + sid
        pltpu.sync_copy(src_h.at[wid], src_v)
        pltpu.sync_copy(dst_h.at[wid], dst_v)
        pltpu.sync_copy(c_h, c_v)
        pltpu.sync_copy(m_h, m_v)
        _owned_copy(sid, lambda o, s: zacc_h.at[pl.ds(o, s)],
                    lambda o, s: acc_sh.at[pl.ds(o, s)])
        plsc.subcore_barrier()
        cvec = c_v[...]
        mvec = m_v[...]
        bufs = ((ra0, rfb0, wo0, gsem0, ssem0),
                (ra1, rfb1, wo1, gsem1, ssem1))

        def gather_start(c, b):
            ra, rfb, wo, gsem, ssem = bufs[b]
            pltpu.async_copy(a_h.at[dst_v.at[c]], ra, gsem)
            pltpu.async_copy(fb_h.at[src_v.at[c]], rfb, gsem)

        def gather_wait(c, b):
            ra, rfb, wo, gsem, ssem = bufs[b]
            pltpu.make_async_copy(a_h.at[dst_v.at[c]], ra, gsem).wait()
            pltpu.make_async_copy(fb_h.at[src_v.at[c]], rfb, gsem).wait()

        def compute(b):
            ra, rfb, wo, gsem, ssem = bufs[b]

            @plsc.parallel_loop(0, K, step=1, unroll=5)
            def edge(ei):
                va = ra[ei]
                vb = rfb[ei, pl.ds(D, 16)]
                ex = jnp.exp(_lrelu(va + vb) - cvec) * mvec
                wo[ei, pl.ds(D, 16)] = ex
                for h in range(H):
                    eh = _splat(ex, h)
                    for j in range(PER_H):
                        off = (h * PER_H + j) * 16
                        wo[ei, pl.ds(off, 16)] = rfb[ei, pl.ds(off, 16)] * eh

        def scatter_start(c, b):
            ra, rfb, wo, gsem, ssem = bufs[b]
            pltpu.async_copy(wo, acc_sh.at[dst_v.at[c]], ssem, add=True)

        def scatter_wait(c, b):
            ra, rfb, wo, gsem, ssem = bufs[b]
            pltpu.make_async_copy(wo, acc_sh.at[dst_v.at[c]], ssem).wait()

        gather_start(0, 0)

        def pair(g, carry):
            c0 = 2 * g
            c1 = c0 + 1
            gather_start(c1, 1)
            gather_wait(c0, 0)

            @pl.when(g > 0)
            def _():
                scatter_wait(c0, 0)

            compute(0)
            scatter_start(c0, 0)

            @pl.when(g < NCH // 2 - 1)
            def _():
                gather_start(c0 + 2, 0)

            gather_wait(c1, 1)

            @pl.when(g > 0)
            def _():
                scatter_wait(c1, 1)

            compute(1)
            scatter_start(c1, 1)
            return carry

        lax.fori_loop(0, NCH // 2, pair, 0)
        scatter_wait(NCH - 2, 0)
        scatter_wait(NCH - 1, 1)
        plsc.subcore_barrier()
        _owned_copy(sid, lambda o, s: acc_sh.at[pl.ds(o, s)],
                    lambda o, s: out_h.at[cid, pl.ds(o, s)])

    return _sc_edge


_sc_edge_l1 = _make_sc_edge(128, 8)
_sc_edge_l2 = _make_sc_edge(64, 1)


def _tc_linear(x, Wt, Wab):
    n, _ = x.shape
    dout = Wt.shape[1]

    def body(x_ref, w_ref, wab_ref, s_ref, m_ref, fb_ref):
        h = jnp.dot(x_ref[...], w_ref[...], preferred_element_type=jnp.float32)
        s = jnp.dot(h, wab_ref[...], preferred_element_type=jnp.float32)
        s_ref[...] = s
        m_ref[...] = jnp.max(s, axis=0, keepdims=True)
        fb_ref[...] = jnp.concatenate([h, s[:, 16:]], axis=1)

    return pl.pallas_call(
        body,
        out_shape=[jax.ShapeDtypeStruct((n, 32), jnp.float32),
                   jax.ShapeDtypeStruct((1, 32), jnp.float32),
                   jax.ShapeDtypeStruct((n, dout + 16), jnp.float32)],
    )(x, Wt, Wab)


def _tc_mid(p, S1, c_row, m_row, b1_row, W2t, Wab2, R, fb1):
    def body(p_ref, s1_ref, c_ref, m_ref, b_ref, w2_ref, wab_ref, r_ref,
             fb1_ref, s_ref, m2_ref, fb_ref):
        s1 = s1_ref[...]
        a = s1[:, :16]
        b = s1[:, 16:]
        ex = jnp.exp(_lrelu(a + b) - c_ref[...]) * m_ref[...]
        acc = p_ref[0] + p_ref[1]
        den16 = acc[:, 128:] + ex
        exx = jnp.dot(ex, r_ref[...], preferred_element_type=jnp.float32)
        denx = jnp.dot(den16, r_ref[...], preferred_element_type=jnp.float32)
        h1 = fb1_ref[...][:, :128]
        nom = acc[:, :128] + exx * h1
        o = nom / (denx + 1e-16) + b_ref[...]
        o = jnp.where(o > 0, o, jnp.exp(o) - 1.0)
        h2 = jnp.dot(o, w2_ref[...], preferred_element_type=jnp.float32)
        s2 = jnp.dot(h2, wab_ref[...], preferred_element_type=jnp.float32)
        s_ref[...] = s2
        m2_ref[...] = jnp.max(s2, axis=0, keepdims=True)
        fb_ref[...] = jnp.concatenate([h2, s2[:, 16:]], axis=1)

    return pl.pallas_call(
        body,
        out_shape=[jax.ShapeDtypeStruct((NN, 32), jnp.float32),
                   jax.ShapeDtypeStruct((1, 32), jnp.float32),
                   jax.ShapeDtypeStruct((NN, 80), jnp.float32)],
    )(p, S1, c_row, m_row, b1_row, W2t, Wab2, R, fb1)


def _tc_final(p, S2, c_row, m_row, fb2, b2_row, R2):
    def body(p_ref, s2_ref, c_ref, m_ref, fb_ref, b_ref, r_ref, o_ref):
        s2 = s2_ref[...]
        a = s2[:, :16]
        b = s2[:, 16:]
        ex = jnp.exp(_lrelu(a + b) - c_ref[...]) * m_ref[...]
        acc = p_ref[0] + p_ref[1]
        den16 = acc[:, 64:] + ex
        exx = jnp.dot(ex, r_ref[...], preferred_element_type=jnp.float32)
        denx = jnp.dot(den16, r_ref[...], preferred_element_type=jnp.float32)
        h2 = fb_ref[...][:, :64]
        nom = acc[:, :64] + exx * h2
        o = nom / (denx + 1e-16) + b_ref[...]
        m = jnp.max(o, axis=1, keepdims=True)
        e = jnp.exp(o - m)
        lse = m + jnp.log(jnp.sum(e, axis=1, keepdims=True))
        o_ref[...] = o - lse

    return pl.pallas_call(
        body,
        out_shape=jax.ShapeDtypeStruct((NN, 64), jnp.float32),
    )(p, S2, c_row, m_row, fb2, b2_row, R2)


def _pack_att(a_src, a_dst, heads, hid):
    eye = jnp.eye(heads, dtype=jnp.float32)
    msrc = (a_src.reshape(heads, hid)[:, :, None] *
            eye[:, None, :]).reshape(heads * hid, heads)
    mdst = (a_dst.reshape(heads, hid)[:, :, None] *
            eye[:, None, :]).reshape(heads * hid, heads)
    pad = jnp.zeros((heads * hid, 16 - heads), jnp.float32)
    return jnp.concatenate([msrc, pad, mdst, pad], axis=1)


def _expander(heads, width):
    r = jnp.zeros((16, heads * width), jnp.float32)
    for h in range(heads):
        r = r.at[h, h * width:(h + 1) * width].set(1.0)
    return r


@jax.jit
def kernel(x, edge_index, W1, a_src1, a_dst1, b1, W2, a_src2, a_dst2, b2):
    src3 = edge_index[0].reshape(NW, NCH, K)
    dst3 = edge_index[1].reshape(NW, NCH, K)
    z144 = jnp.zeros((NN, 144), jnp.float32)
    z80 = jnp.zeros((NN, 80), jnp.float32)
    m1 = jnp.concatenate([jnp.ones(8, jnp.float32), jnp.zeros(8, jnp.float32)])
    m2 = jnp.concatenate([jnp.ones(1, jnp.float32), jnp.zeros(15, jnp.float32)])

    Wab1 = _pack_att(a_src1, a_dst1, 8, 16)
    S1, M1, fb1 = _tc_linear(x, W1.T, Wab1)
    A1 = S1[:, :16]
    c1 = _lrelu(M1[0, :16] + M1[0, 16:]) * m1
    p1 = _sc_edge_l1(src3, dst3, A1, fb1, c1, m1, z144)

    Wab2 = _pack_att(a_src2, a_dst2, 1, 64)
    S2, M2, fb2 = _tc_mid(p1, S1, c1.reshape(1, 16), m1.reshape(1, 16),
                          b1.reshape(1, 128), W2.T, Wab2, _expander(8, 16),
                          fb1)
    A2 = S2[:, :16]
    c2 = _lrelu(M2[0, :16] + M2[0, 16:]) * m2
    p2 = _sc_edge_l2(src3, dst3, A2, fb2, c2, m2, z80)

    return _tc_final(p2, S2, c2.reshape(1, 16), m2.reshape(1, 16), fb2,
                     b2.reshape(1, 64), _expander(1, 64))


# speedup vs baseline: 96.1524x; 1.0790x over previous
import functools

import jax
import jax.numpy as jnp
from jax import lax
from jax.experimental import pallas as pl
from jax.experimental.pallas import tpu as pltpu
from jax.experimental.pallas import tpu_sc as plsc

NN = 10000
EE = 320000
NEGS = 0.2

NC = 2
NS = 16
NW = NC * NS
EW = EE // NW
ROWS = 624

_mesh = plsc.VectorSubcoreMesh(core_axis_name="c", subcore_axis_name="s")
_params = pltpu.CompilerParams(use_tc_tiling_on_sc=False)


def _owned_copy(sid, mk_src, mk_dst):
    off = pl.multiple_of(sid * ROWS, 8)
    pltpu.sync_copy(mk_src(off, ROWS), mk_dst(off, ROWS))

    @pl.when(sid == NS - 1)
    def _():
        pltpu.sync_copy(mk_src(NS * ROWS, NN - NS * ROWS),
                        mk_dst(NS * ROWS, NN - NS * ROWS))


def _lrelu(x):
    return jnp.where(x >= 0, x, x * NEGS)


def _splat(v, lane):
    idx = jnp.full((16, 1), lane, jnp.int32)
    return lax.gather(
        v, idx,
        lax.GatherDimensionNumbers(offset_dims=(), collapsed_slice_dims=(0,),
                                   start_index_map=(0,)),
        (1,), mode=lax.GatherScatterMode.PROMISE_IN_BOUNDS)


def _make_sc_edge(D, H, K, UNROLL):
    PER_H = D // H // 16
    W = D + 16
    NCH = EW // K

    @functools.partial(
        pl.kernel, mesh=_mesh, compiler_params=_params,
        out_type=jax.ShapeDtypeStruct((NC, NN, W), jnp.float32),
        scratch_types=[
            pltpu.VMEM((NCH, K), jnp.int32),
            pltpu.VMEM((NCH, K), jnp.int32),
            pltpu.VMEM((K, 16), jnp.float32),
            pltpu.VMEM((K, W), jnp.float32),
            pltpu.VMEM((K, W), jnp.float32),
            pltpu.VMEM((K, 16), jnp.float32),
            pltpu.VMEM((K, W), jnp.float32),
            pltpu.VMEM((K, W), jnp.float32),
            pltpu.VMEM((16,), jnp.float32),
            pltpu.VMEM((16,), jnp.float32),
            pltpu.VMEM_SHARED((NN, W), jnp.float32),
            pltpu.SemaphoreType.DMA,
            pltpu.SemaphoreType.DMA,
            pltpu.SemaphoreType.DMA,
            pltpu.SemaphoreType.DMA,
        ])
    def _sc_edge(src_h, dst_h, a_h, fb_h, c_h, m_h, zacc_h, out_h,
                 src_v, dst_v, ra0, rfb0, wo0, ra1, rfb1, wo1,
                 c_v, m_v, acc_sh, gsem0, gsem1, ssem0, ssem1):
        cid = lax.axis_index("c")
        sid = lax.axis_index("s")
        wid = cid * NS + sid
        pltpu.sync_copy(src_h.at[wid], src_v)
        pltpu.sync_copy(dst_h.at[wid], dst_v)
        pltpu.sync_copy(c_h, c_v)
        pltpu.sync_copy(m_h, m_v)
        _owned_copy(sid, lambda o, s: zacc_h.at[pl.ds(o, s)],
                    lambda o, s: acc_sh.at[pl.ds(o, s)])
        plsc.subcore_barrier()
        cvec = c_v[...]
        mvec = m_v[...]
        bufs = ((ra0, rfb0, wo0, gsem0, ssem0),
                (ra1, rfb1, wo1, gsem1, ssem1))

        def gather_start(c, b):
            ra, rfb, wo, gsem, ssem = bufs[b]
            pltpu.async_copy(a_h.at[dst_v.at[c]], ra, gsem)
            pltpu.async_copy(fb_h.at[src_v.at[c]], rfb, gsem)

        def gather_wait(c, b):
            ra, rfb, wo, gsem, ssem = bufs[b]
            pltpu.make_async_copy(a_h.at[dst_v.at[c]], ra, gsem).wait()
            pltpu.make_async_copy(fb_h.at[src_v.at[c]], rfb, gsem).wait()

        def compute(b):
            ra, rfb, wo, gsem, ssem = bufs[b]

            @plsc.parallel_loop(0, K, step=1, unroll=UNROLL)
            def edge(ei):
                va = ra[ei]
                vb = rfb[ei, pl.ds(D, 16)]
                ex = jnp.exp(_lrelu(va + vb) - cvec) * mvec
                wo[ei, pl.ds(D, 16)] = ex
                for h in range(H):
                    eh = _splat(ex, h)
                    for j in range(PER_H):
                        off = (h * PER_H + j) * 16
                        wo[ei, pl.ds(off, 16)] = rfb[ei, pl.ds(off, 16)] * eh

        def scatter_start(c, b):
            ra, rfb, wo, gsem, ssem = bufs[b]
            pltpu.async_copy(wo, acc_sh.at[dst_v.at[c]], ssem, add=True)

        def scatter_wait(c, b):
            ra, rfb, wo, gsem, ssem = bufs[b]
            pltpu.make_async_copy(wo, acc_sh.at[dst_v.at[c]], ssem).wait()

        gather_start(0, 0)

        def pair(g, carry):
            c0 = 2 * g
            c1 = c0 + 1
            gather_start(c1, 1)
            gather_wait(c0, 0)

            @pl.when(g > 0)
            def _():
                scatter_wait(c0, 0)

            compute(0)
            scatter_start(c0, 0)

            @pl.when(g < NCH // 2 - 1)
            def _():
                gather_start(c0 + 2, 0)

            gather_wait(c1, 1)

            @pl.when(g > 0)
            def _():
                scatter_wait(c1, 1)

            compute(1)
            scatter_start(c1, 1)
            return carry

        lax.fori_loop(0, NCH // 2, pair, 0)
        scatter_wait(NCH - 2, 0)
        scatter_wait(NCH - 1, 1)
        plsc.subcore_barrier()
        _owned_copy(sid, lambda o, s: acc_sh.at[pl.ds(o, s)],
                    lambda o, s: out_h.at[cid, pl.ds(o, s)])

    return _sc_edge


K1, K2 = 25, 40
_sc_edge_l1 = _make_sc_edge(128, 8, K1, 5)
_sc_edge_l2 = _make_sc_edge(64, 1, K2, 8)


def _tc_linear(x, Wt, Wab):
    n, _ = x.shape
    dout = Wt.shape[1]

    def body(x_ref, w_ref, wab_ref, s_ref, m_ref, fb_ref):
        h = jnp.dot(x_ref[...], w_ref[...], preferred_element_type=jnp.float32)
        s = jnp.dot(h, wab_ref[...], preferred_element_type=jnp.float32)
        s_ref[...] = s
        m_ref[...] = jnp.max(s, axis=0, keepdims=True)
        fb_ref[...] = jnp.concatenate([h, s[:, 16:]], axis=1)

    return pl.pallas_call(
        body,
        out_shape=[jax.ShapeDtypeStruct((n, 32), jnp.float32),
                   jax.ShapeDtypeStruct((1, 32), jnp.float32),
                   jax.ShapeDtypeStruct((n, dout + 16), jnp.float32)],
    )(x, Wt, Wab)


def _tc_mid(p, S1, c_row, m_row, b1_row, W2t, Wab2, R, fb1):
    def body(p_ref, s1_ref, c_ref, m_ref, b_ref, w2_ref, wab_ref, r_ref,
             fb1_ref, s_ref, m2_ref, fb_ref):
        s1 = s1_ref[...]
        a = s1[:, :16]
        b = s1[:, 16:]
        ex = jnp.exp(_lrelu(a + b) - c_ref[...]) * m_ref[...]
        acc = p_ref[0] + p_ref[1]
        den16 = acc[:, 128:] + ex
        exx = jnp.dot(ex, r_ref[...], preferred_element_type=jnp.float32)
        denx = jnp.dot(den16, r_ref[...], preferred_element_type=jnp.float32)
        h1 = fb1_ref[...][:, :128]
        nom = acc[:, :128] + exx * h1
        o = nom / (denx + 1e-16) + b_ref[...]
        o = jnp.where(o > 0, o, jnp.exp(o) - 1.0)
        h2 = jnp.dot(o, w2_ref[...], preferred_element_type=jnp.float32)
        s2 = jnp.dot(h2, wab_ref[...], preferred_element_type=jnp.float32)
        s_ref[...] = s2
        m2_ref[...] = jnp.max(s2, axis=0, keepdims=True)
        fb_ref[...] = jnp.concatenate([h2, s2[:, 16:]], axis=1)

    return pl.pallas_call(
        body,
        out_shape=[jax.ShapeDtypeStruct((NN, 32), jnp.float32),
                   jax.ShapeDtypeStruct((1, 32), jnp.float32),
                   jax.ShapeDtypeStruct((NN, 80), jnp.float32)],
    )(p, S1, c_row, m_row, b1_row, W2t, Wab2, R, fb1)


def _tc_final(p, S2, c_row, m_row, fb2, b2_row, R2):
    def body(p_ref, s2_ref, c_ref, m_ref, fb_ref, b_ref, r_ref, o_ref):
        s2 = s2_ref[...]
        a = s2[:, :16]
        b = s2[:, 16:]
        ex = jnp.exp(_lrelu(a + b) - c_ref[...]) * m_ref[...]
        acc = p_ref[0] + p_ref[1]
        den16 = acc[:, 64:] + ex
        exx = jnp.dot(ex, r_ref[...], preferred_element_type=jnp.float32)
        denx = jnp.dot(den16, r_ref[...], preferred_element_type=jnp.float32)
        h2 = fb_ref[...][:, :64]
        nom = acc[:, :64] + exx * h2
        o = nom / (denx + 1e-16) + b_ref[...]
        m = jnp.max(o, axis=1, keepdims=True)
        e = jnp.exp(o - m)
        lse = m + jnp.log(jnp.sum(e, axis=1, keepdims=True))
        o_ref[...] = o - lse

    return pl.pallas_call(
        body,
        out_shape=jax.ShapeDtypeStruct((NN, 64), jnp.float32),
    )(p, S2, c_row, m_row, fb2, b2_row, R2)


def _pack_att(a_src, a_dst, heads, hid):
    eye = jnp.eye(heads, dtype=jnp.float32)
    msrc = (a_src.reshape(heads, hid)[:, :, None] *
            eye[:, None, :]).reshape(heads * hid, heads)
    mdst = (a_dst.reshape(heads, hid)[:, :, None] *
            eye[:, None, :]).reshape(heads * hid, heads)
    pad = jnp.zeros((heads * hid, 16 - heads), jnp.float32)
    return jnp.concatenate([msrc, pad, mdst, pad], axis=1)


def _expander(heads, width):
    r = jnp.zeros((16, heads * width), jnp.float32)
    for h in range(heads):
        r = r.at[h, h * width:(h + 1) * width].set(1.0)
    return r


@jax.jit
def kernel(x, edge_index, W1, a_src1, a_dst1, b1, W2, a_src2, a_dst2, b2):
    src1 = edge_index[0].reshape(NW, EW // K1, K1)
    dst1 = edge_index[1].reshape(NW, EW // K1, K1)
    src2 = edge_index[0].reshape(NW, EW // K2, K2)
    dst2 = edge_index[1].reshape(NW, EW // K2, K2)
    z144 = jnp.zeros((NN, 144), jnp.float32)
    z80 = jnp.zeros((NN, 80), jnp.float32)
    m1 = jnp.concatenate([jnp.ones(8, jnp.float32), jnp.zeros(8, jnp.float32)])
    m2 = jnp.concatenate([jnp.ones(1, jnp.float32), jnp.zeros(15, jnp.float32)])

    Wab1 = _pack_att(a_src1, a_dst1, 8, 16)
    S1, M1, fb1 = _tc_linear(x, W1.T, Wab1)
    A1 = S1[:, :16]
    c1 = _lrelu(M1[0, :16] + M1[0, 16:]) * m1
    p1 = _sc_edge_l1(src1, dst1, A1, fb1, c1, m1, z144)

    Wab2 = _pack_att(a_src2, a_dst2, 1, 64)
    S2, M2, fb2 = _tc_mid(p1, S1, c1.reshape(1, 16), m1.reshape(1, 16),
                          b1.reshape(1, 128), W2.T, Wab2, _expander(8, 16),
                          fb1)
    A2 = S2[:, :16]
    c2 = _lrelu(M2[0, :16] + M2[0, 16:]) * m2
    p2 = _sc_edge_l2(src2, dst2, A2, fb2, c2, m2, z80)

    return _tc_final(p2, S2, c2.reshape(1, 16), m2.reshape(1, 16), fb2,
                     b2.reshape(1, 64), _expander(1, 64))


# speedup vs baseline: 96.3401x; 1.0020x over previous
import functools

import jax
import jax.numpy as jnp
from jax import lax
from jax.experimental import pallas as pl
from jax.experimental.pallas import tpu as pltpu
from jax.experimental.pallas import tpu_sc as plsc

NN = 10000
EE = 320000
NEGS = 0.2

NC = 2
NS = 16
NW = NC * NS
EW = EE // NW
ROWS = 624

_mesh = plsc.VectorSubcoreMesh(core_axis_name="c", subcore_axis_name="s")
_params = pltpu.CompilerParams(use_tc_tiling_on_sc=False)


def _owned_copy(sid, mk_src, mk_dst):
    off = pl.multiple_of(sid * ROWS, 8)
    pltpu.sync_copy(mk_src(off, ROWS), mk_dst(off, ROWS))

    @pl.when(sid == NS - 1)
    def _():
        pltpu.sync_copy(mk_src(NS * ROWS, NN - NS * ROWS),
                        mk_dst(NS * ROWS, NN - NS * ROWS))


def _lrelu(x):
    return jnp.where(x >= 0, x, x * NEGS)


def _splat(v, lane):
    idx = jnp.full((16, 1), lane, jnp.int32)
    return lax.gather(
        v, idx,
        lax.GatherDimensionNumbers(offset_dims=(), collapsed_slice_dims=(0,),
                                   start_index_map=(0,)),
        (1,), mode=lax.GatherScatterMode.PROMISE_IN_BOUNDS)


def _make_sc_edge(D, H, K, UNROLL):
    PER_H = D // H // 16
    W = D + 16
    NCH = EW // K

    @functools.partial(
        pl.kernel, mesh=_mesh, compiler_params=_params,
        out_type=jax.ShapeDtypeStruct((NC, NN, W), jnp.float32),
        scratch_types=[
            pltpu.VMEM((NCH, K), jnp.int32),
            pltpu.VMEM((NCH, K), jnp.int32),
            pltpu.VMEM((K, 16), jnp.float32),
            pltpu.VMEM((K, W), jnp.float32),
            pltpu.VMEM((K, W), jnp.float32),
            pltpu.VMEM((K, 16), jnp.float32),
            pltpu.VMEM((K, W), jnp.float32),
            pltpu.VMEM((K, W), jnp.float32),
            pltpu.VMEM((16,), jnp.float32),
            pltpu.VMEM((16,), jnp.float32),
            pltpu.VMEM_SHARED((NN, W), jnp.float32),
            pltpu.SemaphoreType.DMA,
            pltpu.SemaphoreType.DMA,
            pltpu.SemaphoreType.DMA,
            pltpu.SemaphoreType.DMA,
        ])
    def _sc_edge(src_h, dst_h, a_h, fb_h, c_h, m_h, zacc_h, out_h,
                 src_v, dst_v, ra0, rfb0, wo0, ra1, rfb1, wo1,
                 c_v, m_v, acc_sh, gsem0, gsem1, ssem0, ssem1):
        cid = lax.axis_index("c")
        sid = lax.axis_index("s")
        wid = cid * NS + sid
        pltpu.sync_copy(src_h.at[wid], src_v)
        pltpu.sync_copy(dst_h.at[wid], dst_v)
        pltpu.sync_copy(c_h, c_v)
        pltpu.sync_copy(m_h, m_v)
        _owned_copy(sid, lambda o, s: zacc_h.at[pl.ds(o, s)],
                    lambda o, s: acc_sh.at[pl.ds(o, s)])
        plsc.subcore_barrier()
        cvec = c_v[...]
        mvec = m_v[...]
        bufs = ((ra0, rfb0, wo0, gsem0, ssem0),
                (ra1, rfb1, wo1, gsem1, ssem1))

        def gather_start(c, b):
            ra, rfb, wo, gsem, ssem = bufs[b]
            pltpu.async_copy(a_h.at[dst_v.at[c]], ra, gsem)
            pltpu.async_copy(fb_h.at[src_v.at[c]], rfb, gsem)

        def gather_wait(c, b):
            ra, rfb, wo, gsem, ssem = bufs[b]
            pltpu.make_async_copy(a_h.at[dst_v.at[c]], ra, gsem).wait()
            pltpu.make_async_copy(fb_h.at[src_v.at[c]], rfb, gsem).wait()

        def compute(b):
            ra, rfb, wo, gsem, ssem = bufs[b]

            @plsc.parallel_loop(0, K, step=1, unroll=UNROLL)
            def edge(ei):
                va = ra[ei]
                vb = rfb[ei, pl.ds(D, 16)]
                ex = jnp.exp(_lrelu(va + vb) - cvec) * mvec
                wo[ei, pl.ds(D, 16)] = ex
                for h in range(H):
                    eh = _splat(ex, h)
                    for j in range(PER_H):
                        off = (h * PER_H + j) * 16
                        wo[ei, pl.ds(off, 16)] = rfb[ei, pl.ds(off, 16)] * eh

        def scatter_start(c, b):
            ra, rfb, wo, gsem, ssem = bufs[b]
            pltpu.async_copy(wo, acc_sh.at[dst_v.at[c]], ssem, add=True)

        def scatter_wait(c, b):
            ra, rfb, wo, gsem, ssem = bufs[b]
            pltpu.make_async_copy(wo, acc_sh.at[dst_v.at[c]], ssem).wait()

        gather_start(0, 0)

        def pair(g, carry):
            c0 = 2 * g
            c1 = c0 + 1
            gather_start(c1, 1)
            gather_wait(c0, 0)

            @pl.when(g > 0)
            def _():
                scatter_wait(c0, 0)

            compute(0)
            scatter_start(c0, 0)

            @pl.when(g < NCH // 2 - 1)
            def _():
                gather_start(c0 + 2, 0)

            gather_wait(c1, 1)

            @pl.when(g > 0)
            def _():
                scatter_wait(c1, 1)

            compute(1)
            scatter_start(c1, 1)
            return carry

        lax.fori_loop(0, NCH // 2, pair, 0)
        scatter_wait(NCH - 2, 0)
        scatter_wait(NCH - 1, 1)
        plsc.subcore_barrier()
        _owned_copy(sid, lambda o, s: acc_sh.at[pl.ds(o, s)],
                    lambda o, s: out_h.at[cid, pl.ds(o, s)])

    return _sc_edge


K1, K2 = 25, 40
_sc_edge_l1 = _make_sc_edge(128, 8, K1, 25)
_sc_edge_l2 = _make_sc_edge(64, 1, K2, 40)


def _tc_linear(x, Wt, Wab):
    n, _ = x.shape
    dout = Wt.shape[1]

    def body(x_ref, w_ref, wab_ref, s_ref, m_ref, fb_ref):
        h = jnp.dot(x_ref[...], w_ref[...], preferred_element_type=jnp.float32)
        s = jnp.dot(h, wab_ref[...], preferred_element_type=jnp.float32)
        s_ref[...] = s
        m_ref[...] = jnp.max(s, axis=0, keepdims=True)
        fb_ref[...] = jnp.concatenate([h, s[:, 16:]], axis=1)

    return pl.pallas_call(
        body,
        out_shape=[jax.ShapeDtypeStruct((n, 32), jnp.float32),
                   jax.ShapeDtypeStruct((1, 32), jnp.float32),
                   jax.ShapeDtypeStruct((n, dout + 16), jnp.float32)],
    )(x, Wt, Wab)


def _tc_mid(p, S1, c_row, m_row, b1_row, W2t, Wab2, R, fb1):
    def body(p_ref, s1_ref, c_ref, m_ref, b_ref, w2_ref, wab_ref, r_ref,
             fb1_ref, s_ref, m2_ref, fb_ref):
        s1 = s1_ref[...]
        a = s1[:, :16]
        b = s1[:, 16:]
        ex = jnp.exp(_lrelu(a + b) - c_ref[...]) * m_ref[...]
        acc = p_ref[0] + p_ref[1]
        den16 = acc[:, 128:] + ex
        exx = jnp.dot(ex, r_ref[...], preferred_element_type=jnp.float32)
        denx = jnp.dot(den16, r_ref[...], preferred_element_type=jnp.float32)
        h1 = fb1_ref[...][:, :128]
        nom = acc[:, :128] + exx * h1
        o = nom / (denx + 1e-16) + b_ref[...]
        o = jnp.where(o > 0, o, jnp.exp(o) - 1.0)
        h2 = jnp.dot(o, w2_ref[...], preferred_element_type=jnp.float32)
        s2 = jnp.dot(h2, wab_ref[...], preferred_element_type=jnp.float32)
        s_ref[...] = s2
        m2_ref[...] = jnp.max(s2, axis=0, keepdims=True)
        fb_ref[...] = jnp.concatenate([h2, s2[:, 16:]], axis=1)

    return pl.pallas_call(
        body,
        out_shape=[jax.ShapeDtypeStruct((NN, 32), jnp.float32),
                   jax.ShapeDtypeStruct((1, 32), jnp.float32),
                   jax.ShapeDtypeStruct((NN, 80), jnp.float32)],
    )(p, S1, c_row, m_row, b1_row, W2t, Wab2, R, fb1)


def _tc_final(p, S2, c_row, m_row, fb2, b2_row, R2):
    def body(p_ref, s2_ref, c_ref, m_ref, fb_ref, b_ref, r_ref, o_ref):
        s2 = s2_ref[...]
        a = s2[:, :16]
        b = s2[:, 16:]
        ex = jnp.exp(_lrelu(a + b) - c_ref[...]) * m_ref[...]
        acc = p_ref[0] + p_ref[1]
        den16 = acc[:, 64:] + ex
        exx = jnp.dot(ex, r_ref[...], preferred_element_type=jnp.float32)
        denx = jnp.dot(den16, r_ref[...], preferred_element_type=jnp.float32)
        h2 = fb_ref[...][:, :64]
        nom = acc[:, :64] + exx * h2
        o = nom / (denx + 1e-16) + b_ref[...]
        m = jnp.max(o, axis=1, keepdims=True)
        e = jnp.exp(o - m)
        lse = m + jnp.log(jnp.sum(e, axis=1, keepdims=True))
        o_ref[...] = o - lse

    return pl.pallas_call(
        body,
        out_shape=jax.ShapeDtypeStruct((NN, 64), jnp.float32),
    )(p, S2, c_row, m_row, fb2, b2_row, R2)


def _pack_att(a_src, a_dst, heads, hid):
    eye = jnp.eye(heads, dtype=jnp.float32)
    msrc = (a_src.reshape(heads, hid)[:, :, None] *
            eye[:, None, :]).reshape(heads * hid, heads)
    mdst = (a_dst.reshape(heads, hid)[:, :, None] *
            eye[:, None, :]).reshape(heads * hid, heads)
    pad = jnp.zeros((heads * hid, 16 - heads), jnp.float32)
    return jnp.concatenate([msrc, pad, mdst, pad], axis=1)


def _expander(heads, width):
    r = jnp.zeros((16, heads * width), jnp.float32)
    for h in range(heads):
        r = r.at[h, h * width:(h + 1) * width].set(1.0)
    return r


@jax.jit
def kernel(x, edge_index, W1, a_src1, a_dst1, b1, W2, a_src2, a_dst2, b2):
    src1 = edge_index[0].reshape(NW, EW // K1, K1)
    dst1 = edge_index[1].reshape(NW, EW // K1, K1)
    src2 = edge_index[0].reshape(NW, EW // K2, K2)
    dst2 = edge_index[1].reshape(NW, EW // K2, K2)
    z144 = jnp.zeros((NN, 144), jnp.float32)
    z80 = jnp.zeros((NN, 80), jnp.float32)
    m1 = jnp.concatenate([jnp.ones(8, jnp.float32), jnp.zeros(8, jnp.float32)])
    m2 = jnp.concatenate([jnp.ones(1, jnp.float32), jnp.zeros(15, jnp.float32)])

    Wab1 = _pack_att(a_src1, a_dst1, 8, 16)
    S1, M1, fb1 = _tc_linear(x, W1.T, Wab1)
    A1 = S1[:, :16]
    c1 = _lrelu(M1[0, :16] + M1[0, 16:]) * m1
    p1 = _sc_edge_l1(src1, dst1, A1, fb1, c1, m1, z144)

    Wab2 = _pack_att(a_src2, a_dst2, 1, 64)
    S2, M2, fb2 = _tc_mid(p1, S1, c1.reshape(1, 16), m1.reshape(1, 16),
                          b1.reshape(1, 128), W2.T, Wab2, _expander(8, 16),
                          fb1)
    A2 = S2[:, :16]
    c2 = _lrelu(M2[0, :16] + M2[0, 16:]) * m2
    p2 = _sc_edge_l2(src2, dst2, A2, fb2, c2, m2, z80)

    return _tc_final(p2, S2, c2.reshape(1, 16), m2.reshape(1, 16), fb2,
                     b2.reshape(1, 64), _expander(1, 64))
